# baseline (speedup 1.0000x reference)
import functools

import jax
import jax.numpy as jnp
from jax.experimental import pallas as pl
from jax.experimental.pallas import tpu as pltpu


_NV = 10000
_NB = 5000
_NBP = 5120
_D = 128

_BK1 = 400
_BM = 1000


def _pass1_kernel(a_ref, x0_ref, s_ref, de_ref, a16_ref, z1_ref, acc_ref,
                  *, nk):
    k = pl.program_id(0)

    @pl.when(k == 0)
    def _():
        acc_ref[...] = jnp.zeros_like(acc_ref)

    a16 = a_ref[...].astype(jnp.bfloat16)
    a16p = jnp.concatenate(
        [a16, jnp.zeros((a16.shape[0], _NBP - _NB), jnp.bfloat16)], axis=1)
    a16_ref[...] = a16p

    w = (s_ref[...] * x0_ref[...]).astype(jnp.bfloat16)
    acc_ref[...] += jax.lax.dot(w.T, a16p,
                                preferred_element_type=jnp.float32)

    @pl.when(k == nk - 1)
    def _():
        z1_ref[...] = (acc_ref[...] * de_ref[...]).astype(jnp.bfloat16).T


def _pass1(a, x0, s, de_row, *, interpret=False):
    nk = _NV // _BK1
    return pl.pallas_call(
        functools.partial(_pass1_kernel, nk=nk),
        grid=(nk,),
        in_specs=[
            pl.BlockSpec((_BK1, _NB), lambda k: (k, 0)),
            pl.BlockSpec((_BK1, _D), lambda k: (k, 0)),
            pl.BlockSpec((_BK1, 1), lambda k: (k, 0)),
            pl.BlockSpec((1, _NBP), lambda k: (0, 0)),
        ],
        out_specs=[
            pl.BlockSpec((_BK1, _NBP), lambda k: (k, 0)),
            pl.BlockSpec((_NBP, _D), lambda k: (0, 0)),
        ],
        out_shape=[
            jax.ShapeDtypeStruct((_NV, _NBP), jnp.bfloat16),
            jax.ShapeDtypeStruct((_NBP, _D), jnp.bfloat16),
        ],
        scratch_shapes=[pltpu.VMEM((_D, _NBP), jnp.float32)],
        interpret=interpret,
    )(a, x0, s, de_row)


def _pass2_kernel(a16_ref, z1_ref, s_ref, dv_ref, de_ref, x1_ref, z2_ref,
                  acc_ref, *, nm):
    m = pl.program_id(0)

    @pl.when(m == 0)
    def _():
        acc_ref[...] = jnp.zeros_like(acc_ref)

    x1 = dv_ref[...] * jax.lax.dot(a16_ref[...], z1_ref[...],
                                   preferred_element_type=jnp.float32)
    x1_ref[...] = x1

    w = (s_ref[...] * x1).astype(jnp.bfloat16)
    acc_ref[...] += jax.lax.dot(w.T, a16_ref[...],
                                preferred_element_type=jnp.float32)

    @pl.when(m == nm - 1)
    def _():
        z2_ref[...] = (acc_ref[...] * de_ref[...]).astype(jnp.bfloat16).T


def _pass2(a16, z1, s, dv, de_row, *, interpret=False):
    nm = _NV // _BM
    return pl.pallas_call(
        functools.partial(_pass2_kernel, nm=nm),
        grid=(nm,),
        in_specs=[
            pl.BlockSpec((_BM, _NBP), lambda m: (m, 0)),
            pl.BlockSpec((_NBP, _D), lambda m: (0, 0)),
            pl.BlockSpec((_BM, 1), lambda m: (m, 0)),
            pl.BlockSpec((_BM, 1), lambda m: (m, 0)),
            pl.BlockSpec((1, _NBP), lambda m: (0, 0)),
        ],
        out_specs=[
            pl.BlockSpec((_BM, _D), lambda m: (m, 0)),
            pl.BlockSpec((_NBP, _D), lambda m: (0, 0)),
        ],
        out_shape=[
            jax.ShapeDtypeStruct((_NV, _D), jnp.float32),
            jax.ShapeDtypeStruct((_NBP, _D), jnp.bfloat16),
        ],
        scratch_shapes=[pltpu.VMEM((_D, _NBP), jnp.float32)],
        interpret=interpret,
    )(a16, z1, s, dv, de_row)


def _pass3_kernel(a16_ref, z2_ref, dv_ref, x0_ref, x1_ref, o_ref):
    x2 = dv_ref[...] * jax.lax.dot(a16_ref[...], z2_ref[...],
                                   preferred_element_type=jnp.float32)
    o_ref[...] = (x0_ref[...] + x1_ref[...] + x2) * (1.0 / 3.0)


def _pass3(a16, z2, dv, x0, x1, *, interpret=False):
    nm = _NV // _BM
    return pl.pallas_call(
        _pass3_kernel,
        grid=(nm,),
        in_specs=[
            pl.BlockSpec((_BM, _NBP), lambda m: (m, 0)),
            pl.BlockSpec((_NBP, _D), lambda m: (0, 0)),
            pl.BlockSpec((_BM, 1), lambda m: (m, 0)),
            pl.BlockSpec((_BM, _D), lambda m: (m, 0)),
            pl.BlockSpec((_BM, _D), lambda m: (m, 0)),
        ],
        out_specs=pl.BlockSpec((_BM, _D), lambda m: (m, 0)),
        out_shape=jax.ShapeDtypeStruct((_NV, _D), jnp.float32),
        interpret=interpret,
    )(a16, z2, dv, x0, x1)


def _run(users_embedding, product_embedding, adj_matrix, degreeV_matrix,
         degreeE_matrix, gate_user, gate_product, interpret=False):
    num_users = users_embedding.shape[0]
    x0 = jnp.concatenate([users_embedding, product_embedding], axis=0)
    dv = degreeV_matrix[:, None]
    de_row = jnp.pad(degreeE_matrix, (0, _NBP - _NB))[None, :]
    gates = jnp.concatenate([
        jnp.broadcast_to(gate_user, (num_users, 1)),
        jnp.broadcast_to(gate_product, (_NV - num_users, 1)),
    ])
    s = dv * gates

    a16, z1 = _pass1(adj_matrix, x0, s, de_row, interpret=interpret)
    x1, z2 = _pass2(a16, z1, s, dv, de_row, interpret=interpret)
    out = _pass3(a16, z2, dv, x0, x1, interpret=interpret)
    return out[:num_users], out[num_users:]


def kernel(users_embedding, product_embedding, adj_matrix, degreeV_matrix,
           degreeE_matrix, gate_user, gate_product):
    return _run(users_embedding, product_embedding, adj_matrix,
                degreeV_matrix, degreeE_matrix, gate_user, gate_product)


# speedup vs baseline: 1.0175x; 1.0175x over previous
import functools

import jax
import jax.numpy as jnp
from jax.experimental import pallas as pl
from jax.experimental.pallas import tpu as pltpu


_NV = 10000
_NB = 5000
_NBP = 5120
_D = 128

_BK1 = 400
_BM = 1000


def _pass1_kernel(a_ref, x0_ref, s_ref, de_ref, a16_ref, z1_ref, acc_ref,
                  *, nk):
    k = pl.program_id(0)

    @pl.when(k == 0)
    def _():
        acc_ref[...] = jnp.zeros_like(acc_ref)

    a16 = a_ref[...].astype(jnp.bfloat16)
    a16p = jnp.concatenate(
        [a16, jnp.zeros((a16.shape[0], _NBP - _NB), jnp.bfloat16)], axis=1)
    a16_ref[...] = a16p

    w = (s_ref[...] * x0_ref[...]).astype(jnp.bfloat16)
    acc_ref[...] += jax.lax.dot(w.T, a16p,
                                preferred_element_type=jnp.float32)

    @pl.when(k == nk - 1)
    def _():
        z1_ref[...] = (acc_ref[...] * de_ref[...]).astype(jnp.bfloat16).T


def _pass1(a, x0, s, de_row, *, interpret=False):
    nk = _NV // _BK1
    return pl.pallas_call(
        functools.partial(_pass1_kernel, nk=nk),
        grid=(nk,),
        in_specs=[
            pl.BlockSpec((_BK1, _NB), lambda k: (k, 0)),
            pl.BlockSpec((_BK1, _D), lambda k: (k, 0)),
            pl.BlockSpec((_BK1, 1), lambda k: (k, 0)),
            pl.BlockSpec((1, _NBP), lambda k: (0, 0)),
        ],
        out_specs=[
            pl.BlockSpec((_BK1, _NBP), lambda k: (k, 0)),
            pl.BlockSpec((_NBP, _D), lambda k: (0, 0)),
        ],
        out_shape=[
            jax.ShapeDtypeStruct((_NV, _NBP), jnp.bfloat16),
            jax.ShapeDtypeStruct((_NBP, _D), jnp.bfloat16),
        ],
        scratch_shapes=[pltpu.VMEM((_D, _NBP), jnp.float32)],
        interpret=interpret,
    )(a, x0, s, de_row)


def _pass23_kernel(a16_ref, z1_ref, s_ref, dv2_ref, dv3_ref, x0_ref, de_ref,
                   o_ref, acc_ref, z2_ref, x1s_ref, *, nm):
    m = pl.program_id(0)

    @pl.when(m == 0)
    def _():
        acc_ref[...] = jnp.zeros_like(acc_ref)

    @pl.when(m < nm)
    def _():
        x1 = dv2_ref[...] * jax.lax.dot(a16_ref[...], z1_ref[...],
                                        preferred_element_type=jnp.float32)
        x1s_ref[m] = x1
        w = (s_ref[...] * x1).astype(jnp.bfloat16)
        acc_ref[...] += jax.lax.dot(w.T, a16_ref[...],
                                    preferred_element_type=jnp.float32)

    @pl.when(m == nm - 1)
    def _():
        z2_ref[...] = (acc_ref[...] * de_ref[...]).astype(jnp.bfloat16).T

    @pl.when(m >= nm)
    def _():
        x2 = dv3_ref[...] * jax.lax.dot(a16_ref[...], z2_ref[...],
                                        preferred_element_type=jnp.float32)
        o_ref[...] = (x0_ref[...] + x1s_ref[m - nm] + x2) * (1.0 / 3.0)


def _pass23(a16, z1, s, dv, de_row, x0, *, interpret=False):
    nm = _NV // _BM
    return pl.pallas_call(
        functools.partial(_pass23_kernel, nm=nm),
        grid=(2 * nm,),
        in_specs=[
            pl.BlockSpec((_BM, _NBP), lambda m: (m % nm, 0)),
            pl.BlockSpec((_NBP, _D), lambda m: (0, 0)),
            pl.BlockSpec((_BM, 1), lambda m: (jnp.minimum(m, nm - 1), 0)),
            pl.BlockSpec((_BM, 1), lambda m: (jnp.minimum(m, nm - 1), 0)),
            pl.BlockSpec((_BM, 1), lambda m: (jnp.maximum(m - nm, 0), 0)),
            pl.BlockSpec((_BM, _D), lambda m: (jnp.maximum(m - nm, 0), 0)),
            pl.BlockSpec((1, _NBP), lambda m: (0, 0)),
        ],
        out_specs=pl.BlockSpec(
            (_BM, _D), lambda m: (jnp.maximum(m - nm, 0), 0)),
        out_shape=jax.ShapeDtypeStruct((_NV, _D), jnp.float32),
        scratch_shapes=[
            pltpu.VMEM((_D, _NBP), jnp.float32),
            pltpu.VMEM((_NBP, _D), jnp.bfloat16),
            pltpu.VMEM((_NV // _BM, _BM, _D), jnp.float32),
        ],
        interpret=interpret,
    )(a16, z1, s, dv, dv, x0, de_row)


def _run(users_embedding, product_embedding, adj_matrix, degreeV_matrix,
         degreeE_matrix, gate_user, gate_product, interpret=False):
    num_users = users_embedding.shape[0]
    x0 = jnp.concatenate([users_embedding, product_embedding], axis=0)
    dv = degreeV_matrix[:, None]
    de_row = jnp.pad(degreeE_matrix, (0, _NBP - _NB))[None, :]
    gates = jnp.concatenate([
        jnp.broadcast_to(gate_user, (num_users, 1)),
        jnp.broadcast_to(gate_product, (_NV - num_users, 1)),
    ])
    s = dv * gates

    a16, z1 = _pass1(adj_matrix, x0, s, de_row, interpret=interpret)
    out = _pass23(a16, z1, s, dv, de_row, x0, interpret=interpret)
    return out[:num_users], out[num_users:]


def kernel(users_embedding, product_embedding, adj_matrix, degreeV_matrix,
           degreeE_matrix, gate_user, gate_product):
    return _run(users_embedding, product_embedding, adj_matrix,
                degreeV_matrix, degreeE_matrix, gate_user, gate_product)


# speedup vs baseline: 1.0183x; 1.0008x over previous
import functools

import jax
import jax.numpy as jnp
from jax.experimental import pallas as pl
from jax.experimental.pallas import tpu as pltpu


_NV = 10000
_NB = 5000
_NBP = 5120
_D = 128

_BK1 = 400
_BM = 1000


def _pass1_kernel(a_ref, x0_ref, s_ref, de_ref, a16_ref, z1_ref, acc_ref,
                  *, nk):
    k = pl.program_id(0)

    @pl.when(k == 0)
    def _():
        acc_ref[...] = jnp.zeros_like(acc_ref)

    a16 = a_ref[...].astype(jnp.bfloat16)
    a16p = jnp.concatenate(
        [a16, jnp.zeros((a16.shape[0], _NBP - _NB), jnp.bfloat16)], axis=1)
    a16_ref[...] = a16p

    w = (s_ref[...] * x0_ref[...]).astype(jnp.bfloat16)
    acc_ref[...] += jax.lax.dot(w.T, a16p,
                                preferred_element_type=jnp.float32)

    @pl.when(k == nk - 1)
    def _():
        z1_ref[...] = (acc_ref[...] * de_ref[...]).astype(jnp.bfloat16).T


def _pass1(a, x0, s, de_row, *, interpret=False):
    nk = _NV // _BK1
    return pl.pallas_call(
        functools.partial(_pass1_kernel, nk=nk),
        grid=(nk,),
        in_specs=[
            pl.BlockSpec((_BK1, _NB), lambda k: (k, 0)),
            pl.BlockSpec((_BK1, _D), lambda k: (k, 0)),
            pl.BlockSpec((_BK1, 1), lambda k: (k, 0)),
            pl.BlockSpec((1, _NBP), lambda k: (0, 0)),
        ],
        out_specs=[
            pl.BlockSpec((_BK1, _NBP), lambda k: (k, 0)),
            pl.BlockSpec((_NBP, _D), lambda k: (0, 0)),
        ],
        out_shape=[
            jax.ShapeDtypeStruct((_NV, _NBP), jnp.bfloat16),
            jax.ShapeDtypeStruct((_NBP, _D), jnp.bfloat16),
        ],
        scratch_shapes=[pltpu.VMEM((_D, _NBP), jnp.float32)],
        interpret=interpret,
    )(a, x0, s, de_row)


def _pass23_kernel(a16_ref, z1_ref, s_ref, dv_ref, x0_ref, de_ref,
                   o_ref, acc_ref, z2_ref, x1s_ref, *, nm):
    m = pl.program_id(0)

    @pl.when(m == 0)
    def _():
        acc_ref[...] = jnp.zeros_like(acc_ref)

    @pl.when(m < nm)
    def _():
        x1 = dv_ref[...] * jax.lax.dot(a16_ref[...], z1_ref[...],
                                       preferred_element_type=jnp.float32)
        x1s_ref[m] = x1.astype(jnp.bfloat16)
        w = (s_ref[...] * x1).astype(jnp.bfloat16)
        acc_ref[...] += jax.lax.dot(w.T, a16_ref[...],
                                    preferred_element_type=jnp.float32)

    @pl.when(m == nm - 1)
    def _():
        z2_ref[...] = (acc_ref[...] * de_ref[...]).astype(jnp.bfloat16).T

    @pl.when(m >= nm)
    def _():
        x2 = dv_ref[...] * jax.lax.dot(a16_ref[...], z2_ref[...],
                                       preferred_element_type=jnp.float32)
        x1 = x1s_ref[m - nm].astype(jnp.float32)
        o_ref[...] = (x0_ref[...] + x1 + x2) * (1.0 / 3.0)


def _pass23(a16, z1, s, dv, de_row, x0, *, interpret=False):
    nm = _NV // _BM
    return pl.pallas_call(
        functools.partial(_pass23_kernel, nm=nm),
        grid=(2 * nm,),
        in_specs=[
            pl.BlockSpec((_BM, _NBP), lambda m: (m % nm, 0)),
            pl.BlockSpec((_NBP, _D), lambda m: (0, 0)),
            pl.BlockSpec((_BM, 1), lambda m: (jnp.minimum(m, nm - 1), 0)),
            pl.BlockSpec((_BM, 1), lambda m: (m % nm, 0)),
            pl.BlockSpec((_BM, _D), lambda m: (jnp.maximum(m - nm, 0), 0)),
            pl.BlockSpec((1, _NBP), lambda m: (0, 0)),
        ],
        out_specs=pl.BlockSpec(
            (_BM, _D), lambda m: (jnp.maximum(m - nm, 0), 0)),
        out_shape=jax.ShapeDtypeStruct((_NV, _D), jnp.float32),
        scratch_shapes=[
            pltpu.VMEM((_D, _NBP), jnp.float32),
            pltpu.VMEM((_NBP, _D), jnp.bfloat16),
            pltpu.VMEM((_NV // _BM, _BM, _D), jnp.bfloat16),
        ],
        interpret=interpret,
    )(a16, z1, s, dv, x0, de_row)


def _run(users_embedding, product_embedding, adj_matrix, degreeV_matrix,
         degreeE_matrix, gate_user, gate_product, interpret=False):
    num_users = users_embedding.shape[0]
    x0 = jnp.concatenate([users_embedding, product_embedding], axis=0)
    dv = degreeV_matrix[:, None]
    de_row = jnp.pad(degreeE_matrix, (0, _NBP - _NB))[None, :]
    gates = jnp.concatenate([
        jnp.broadcast_to(gate_user, (num_users, 1)),
        jnp.broadcast_to(gate_product, (_NV - num_users, 1)),
    ])
    s = dv * gates

    a16, z1 = _pass1(adj_matrix, x0, s, de_row, interpret=interpret)
    out = _pass23(a16, z1, s, dv, de_row, x0, interpret=interpret)
    return out[:num_users], out[num_users:]


def kernel(users_embedding, product_embedding, adj_matrix, degreeV_matrix,
           degreeE_matrix, gate_user, gate_product):
    return _run(users_embedding, product_embedding, adj_matrix,
                degreeV_matrix, degreeE_matrix, gate_user, gate_product)
